# baseline (speedup 1.0000x reference)
import jax
import jax.numpy as jnp
from jax import lax
from jax.experimental import pallas as pl
from jax.experimental.pallas import tpu as pltpu
from jax.experimental.pallas import tpu_sc as plsc

_B, _D, _H, _W, _M = 8, 10, 188, 188, 500
_HW = _H * _W
_MP = 512
_NT = 16
_NC = 2
_NW = _NT * _NC
_CHUNK = 128
_DH = _D // 2
_NCHUNKS = _DH * _B * (_MP // _CHUNK)
_CPT = _NCHUNKS // _NW
_NV = _CHUNK // 16
_MASKV = _B * _MP // 16


def _make_body(d0):
    def _sc_loss_body(outflat, indflat, maskflat, tgtflat, part, out,
                      ind_v, mask_v, idxp_v, idxt_v, pred_v, tgt_v,
                      red_v, sum_v, psem, tsem):
        core = lax.axis_index("c")
        sub = lax.axis_index("s")
        wid = core * _NT + sub

        pltpu.sync_copy(indflat, ind_v)
        pltpu.sync_copy(maskflat, mask_v)

        lanes = lax.iota(jnp.int32, 16)

        handles = []
        for k in range(_CPT):
            c = wid * _CPT + k
            dl = c // 32
            r = c % 32
            b = r // 4
            mc = r % 4
            ioff = b * _MP + mc * _CHUNK
            pbase = dl * (_H * _B * _W)
            for j in range(_NV):
                iv = ind_v[pl.ds(ioff + j * 16, 16)]
                idxp_v[k, pl.ds(j * 16, 16)] = iv + pbase
                mvec = mc * _CHUNK + j * 16 + lanes
                mclamp = jnp.minimum(mvec, _M - 1)
                idxt_v[k, pl.ds(j * 16, 16)] = (b * _M + mclamp) * _D + (d0 + dl)
            hp = pltpu.async_copy(outflat.at[idxp_v.at[k]], pred_v.at[k], psem)
            ht = pltpu.async_copy(tgtflat.at[idxt_v.at[k]], tgt_v.at[k], tsem)
            handles.append((hp, ht))

        for hp, ht in handles:
            hp.wait()
            ht.wait()

        acc = jnp.zeros((16,), jnp.float32)
        for k in range(_CPT):
            c = wid * _CPT + k
            dl = c // 32
            r = c % 32
            b = r // 4
            mc = r % 4
            ioff = b * _MP + mc * _CHUNK
            csum = jnp.zeros((16,), jnp.float32)
            for j in range(_NV):
                p = pred_v[k, pl.ds(j * 16, 16)]
                t = tgt_v[k, pl.ds(j * 16, 16)]
                w = mask_v[pl.ds(ioff + j * 16, 16)]
                wm = jnp.where(t == t, w, jnp.float32(0.0))
                csum = csum + jnp.abs(p * wm - t * wm)
            acc = acc + jnp.where(lanes == d0 + dl, jnp.sum(csum),
                                  jnp.float32(0.0))

        red_v[...] = acc
        pltpu.sync_copy(red_v, part.at[wid])
        plsc.subcore_barrier()

        @pl.when(sub == 0)
        def _final():
            pltpu.sync_copy(part.at[pl.ds(core * _NT, _NT)], sum_v)
            tot = jnp.zeros((16,), jnp.float32)
            for i in range(_NT):
                tot = tot + sum_v[i, :]
            mks = jnp.zeros((16,), jnp.float32)
            for i in range(_MASKV):
                mks = mks + mask_v[pl.ds(i * 16, 16)]
            num_v = jnp.full((16,), jnp.sum(mks), jnp.float32)
            denom = jnp.maximum(num_v, jnp.float32(1.0))
            red_v[...] = tot / denom
            pltpu.sync_copy(red_v, out.at[core])

    return _sc_loss_body


def _half_call(body):
    mesh = plsc.VectorSubcoreMesh(core_axis_name="c", subcore_axis_name="s")
    return pl.kernel(
        body,
        out_type=(jax.ShapeDtypeStruct((_NW, 16), jnp.float32),
                  jax.ShapeDtypeStruct((_NC, 16), jnp.float32)),
        mesh=mesh,
        compiler_params=pltpu.CompilerParams(needs_layout_passes=False),
        scratch_types=[
            pltpu.VMEM((_B * _MP,), jnp.int32),
            pltpu.VMEM((_B * _MP,), jnp.float32),
            pltpu.VMEM((_CPT, _CHUNK), jnp.int32),
            pltpu.VMEM((_CPT, _CHUNK), jnp.int32),
            pltpu.VMEM((_CPT, _CHUNK), jnp.float32),
            pltpu.VMEM((_CPT, _CHUNK), jnp.float32),
            pltpu.VMEM((16,), jnp.float32),
            pltpu.VMEM((_NT, 16), jnp.float32),
            pltpu.SemaphoreType.DMA,
            pltpu.SemaphoreType.DMA,
        ],
    )


def kernel(output, mask, ind, target):
    fa = jnp.transpose(output[:, :_DH], (1, 2, 0, 3)).reshape(-1)
    fb = jnp.transpose(output[:, _DH:], (1, 2, 0, 3)).reshape(-1)
    ind32 = ind.astype(jnp.int32)
    h = ind32 // _W
    w = ind32 - h * _W
    base = h * (_B * _W) + jnp.arange(_B, dtype=jnp.int32)[:, None] * _W + w
    indflat = jnp.pad(base, ((0, 0), (0, _MP - _M))).reshape(-1)
    maskflat = jnp.pad(mask.astype(jnp.float32),
                       ((0, 0), (0, _MP - _M))).reshape(-1)
    tgtflat = target.reshape(-1)

    _, ra = _half_call(_make_body(0))(fa, indflat, maskflat, tgtflat)
    _, rb = _half_call(_make_body(_DH))(fb, indflat, maskflat, tgtflat)
    res = ra[0] + ra[1] + rb[0] + rb[1]
    return res[:_D]


# speedup vs baseline: 1.1917x; 1.1917x over previous
import jax
import jax.numpy as jnp
from jax import lax
from jax.experimental import pallas as pl
from jax.experimental.pallas import tpu as pltpu
from jax.experimental.pallas import tpu_sc as plsc

_B, _D, _H, _W, _M = 8, 10, 188, 188, 500
_HW = _H * _W
_MP = 512
_NT = 16
_NC = 2
_NW = _NT * _NC
_CHUNK = 128
_TB = _CHUNK * _D
_NCHUNKS = _D * _B * (_MP // _CHUNK)
_CPT = _NCHUNKS // _NW
_NV = _CHUNK // 16


def _sc_loss_body(outflat, indflat, maskflat, tgtflat, part, out,
                  ind_v, mask_v, idxp_v, pred_v, tgt_v,
                  red_v, sum_v, psem, tsem):
    core = lax.axis_index("c")
    sub = lax.axis_index("s")
    wid = core * _NT + sub

    pltpu.sync_copy(indflat, ind_v)
    pltpu.sync_copy(maskflat, mask_v)

    lanes = lax.iota(jnp.int32, 16)

    handles = []
    for k in range(_CPT):
        c = wid * _CPT + k
        d = c // 32
        r = c % 32
        b = r // 4
        mc = r % 4
        ioff = b * _MP + mc * _CHUNK
        pbase = d * (_H * _B * _W)
        for j in range(_NV):
            iv = ind_v[pl.ds(ioff + j * 16, 16)]
            idxp_v[k, pl.ds(j * 16, 16)] = iv + pbase
        hp = pltpu.async_copy(outflat.at[idxp_v.at[k]], pred_v.at[k], psem)
        ht = pltpu.async_copy(tgtflat.at[pl.ds(ioff * _D, _TB)],
                              tgt_v.at[k], tsem)
        handles.append((hp, ht))

    for hp, ht in handles:
        hp.wait()
        ht.wait()

    acc = jnp.zeros((16,), jnp.float32)
    msum = jnp.float32(0.0)
    dfirst = core * (_D // _NC)
    lanes_d = lanes * _D
    for k in range(_CPT):
        c = wid * _CPT + k
        d = c // 32
        r = c % 32
        b = r // 4
        mc = r % 4
        ioff = b * _MP + mc * _CHUNK
        kvec = jnp.full((16,), k, jnp.int32)
        csum = jnp.zeros((16,), jnp.float32)
        mk = jnp.zeros((16,), jnp.float32)
        for j in range(_NV):
            p = pred_v[k, pl.ds(j * 16, 16)]
            t = plsc.load_gather(tgt_v, [kvec, lanes_d + (j * 16 * _D + d)])
            w = mask_v[pl.ds(ioff + j * 16, 16)]
            wm = jnp.where(t == t, w, jnp.float32(0.0))
            csum = csum + jnp.abs(p * wm - t * wm)
            mk = mk + w
        sval = jnp.sum(csum)
        msum = msum + jnp.where(d == dfirst, jnp.sum(mk), jnp.float32(0.0))
        acc = acc + jnp.where(lanes == d, sval, jnp.float32(0.0))

    acc = acc + jnp.where(lanes == _D, msum, jnp.float32(0.0))

    red_v[...] = acc
    pltpu.sync_copy(red_v, part.at[wid])
    plsc.subcore_barrier()

    @pl.when(sub == 0)
    def _final():
        pltpu.sync_copy(part.at[pl.ds(core * _NT, _NT)], sum_v)
        tot = jnp.zeros((16,), jnp.float32)
        for i in range(_NT):
            tot = tot + sum_v[i, :]
        num_v = jnp.full((16,), tot[_D], jnp.float32)
        denom = jnp.maximum(num_v, jnp.float32(1.0))
        red_v[...] = tot / denom
        pltpu.sync_copy(red_v, out.at[core])


def kernel(output, mask, ind, target):
    outflat = jnp.transpose(output, (1, 2, 0, 3)).reshape(-1)
    ind32 = ind.astype(jnp.int32)
    h = ind32 // _W
    w = ind32 - h * _W
    base = h * (_B * _W) + jnp.arange(_B, dtype=jnp.int32)[:, None] * _W + w
    indflat = jnp.pad(base, ((0, 0), (0, _MP - _M))).reshape(-1)
    maskflat = jnp.pad(mask.astype(jnp.float32),
                       ((0, 0), (0, _MP - _M))).reshape(-1)
    tgtflat = jnp.pad(target, ((0, 0), (0, _MP - _M), (0, 0))).reshape(-1)

    mesh = plsc.VectorSubcoreMesh(core_axis_name="c", subcore_axis_name="s")
    f = pl.kernel(
        _sc_loss_body,
        out_type=(jax.ShapeDtypeStruct((_NW, 16), jnp.float32),
                  jax.ShapeDtypeStruct((_NC, 16), jnp.float32)),
        mesh=mesh,
        compiler_params=pltpu.CompilerParams(needs_layout_passes=False),
        scratch_types=[
            pltpu.VMEM((_B * _MP,), jnp.int32),
            pltpu.VMEM((_B * _MP,), jnp.float32),
            pltpu.VMEM((_CPT, _CHUNK), jnp.int32),
            pltpu.VMEM((_CPT, _CHUNK), jnp.float32),
            pltpu.VMEM((_CPT, _TB), jnp.float32),
            pltpu.VMEM((16,), jnp.float32),
            pltpu.VMEM((_NT, 16), jnp.float32),
            pltpu.SemaphoreType.DMA,
            pltpu.SemaphoreType.DMA,
        ],
    )
    _, res = f(outflat, indflat, maskflat, tgtflat)
    return (res[0, :_D] + res[1, :_D])


# speedup vs baseline: 1.2078x; 1.0135x over previous
import jax
import jax.numpy as jnp
from jax import lax
from jax.experimental import pallas as pl
from jax.experimental.pallas import tpu as pltpu
from jax.experimental.pallas import tpu_sc as plsc

_B, _D, _H, _W, _M = 8, 10, 188, 188, 500
_HW = _H * _W
_MP = 512
_NT = 16
_NC = 2
_NW = _NT * _NC
_CHUNK = 128
_NCHUNKS = _D * _B * (_MP // _CHUNK)
_CPT = _NCHUNKS // _NW
_NV = _CHUNK // 16


def _sc_loss_body(outflat, indflat, maskflat, tgtflat, part, out,
                  ind_v, mask_v, idxp_v, idxt_v, pred_v, tgt_v,
                  red_v, sum_v, psem, tsem):
    core = lax.axis_index("c")
    sub = lax.axis_index("s")
    wid = core * _NT + sub

    pltpu.sync_copy(indflat, ind_v)
    pltpu.sync_copy(maskflat, mask_v)

    lanes = lax.iota(jnp.int32, 16)

    handles = []
    for k in range(_CPT):
        c = wid * _CPT + k
        d = c // 32
        r = c % 32
        b = r // 4
        mc = r % 4
        ioff = b * _MP + mc * _CHUNK
        pbase = d * (_H * _B * _W)
        for j in range(_NV):
            iv = ind_v[pl.ds(ioff + j * 16, 16)]
            idxp_v[k, pl.ds(j * 16, 16)] = iv + pbase
            mvec = mc * _CHUNK + j * 16 + lanes
            mclamp = jnp.minimum(mvec, _M - 1)
            idxt_v[k, pl.ds(j * 16, 16)] = (b * _M + mclamp) * _D + d
        hp = pltpu.async_copy(outflat.at[idxp_v.at[k]], pred_v.at[k], psem)
        ht = pltpu.async_copy(tgtflat.at[idxt_v.at[k]], tgt_v.at[k], tsem)
        handles.append((hp, ht))

    for hp, ht in handles:
        hp.wait()
        ht.wait()

    acc = jnp.zeros((16,), jnp.float32)
    msum = jnp.float32(0.0)
    dfirst = core * (_D // _NC)
    for k in range(_CPT):
        c = wid * _CPT + k
        d = c // 32
        r = c % 32
        b = r // 4
        mc = r % 4
        ioff = b * _MP + mc * _CHUNK
        csum = jnp.zeros((16,), jnp.float32)
        mk = jnp.zeros((16,), jnp.float32)
        for j in range(_NV):
            p = pred_v[k, pl.ds(j * 16, 16)]
            t = tgt_v[k, pl.ds(j * 16, 16)]
            w = mask_v[pl.ds(ioff + j * 16, 16)]
            wm = jnp.where(t == t, w, jnp.float32(0.0))
            csum = csum + jnp.abs(p * wm - t * wm)
            mk = mk + w
        sval = jnp.sum(csum)
        msum = msum + jnp.where(d == dfirst, jnp.sum(mk), jnp.float32(0.0))
        acc = acc + jnp.where(lanes == d, sval, jnp.float32(0.0))

    acc = acc + jnp.where(lanes == _D, msum, jnp.float32(0.0))

    red_v[...] = acc
    pltpu.sync_copy(red_v, part.at[wid])
    plsc.subcore_barrier()

    @pl.when(sub == 0)
    def _final():
        pltpu.sync_copy(part.at[pl.ds(core * _NT, _NT)], sum_v)
        tot = jnp.zeros((16,), jnp.float32)
        for i in range(_NT):
            tot = tot + sum_v[i, :]
        num_v = jnp.full((16,), tot[_D], jnp.float32)
        denom = jnp.maximum(num_v, jnp.float32(1.0))
        red_v[...] = tot / denom
        pltpu.sync_copy(red_v, out.at[core])


def kernel(output, mask, ind, target):
    outflat = jnp.transpose(output, (1, 2, 0, 3)).reshape(-1)
    ind32 = ind.astype(jnp.int32)
    h = ind32 // _W
    w = ind32 - h * _W
    base = h * (_B * _W) + jnp.arange(_B, dtype=jnp.int32)[:, None] * _W + w
    indflat = jnp.pad(base, ((0, 0), (0, _MP - _M))).reshape(-1)
    maskflat = jnp.pad(mask.astype(jnp.float32),
                       ((0, 0), (0, _MP - _M))).reshape(-1)
    tgtflat = target.reshape(-1)

    mesh = plsc.VectorSubcoreMesh(core_axis_name="c", subcore_axis_name="s")
    f = pl.kernel(
        _sc_loss_body,
        out_type=(jax.ShapeDtypeStruct((_NW, 16), jnp.float32),
                  jax.ShapeDtypeStruct((_NC, 16), jnp.float32)),
        mesh=mesh,
        compiler_params=pltpu.CompilerParams(needs_layout_passes=False),
        scratch_types=[
            pltpu.VMEM((_B * _MP,), jnp.int32),
            pltpu.VMEM((_B * _MP,), jnp.float32),
            pltpu.VMEM((_CPT, _CHUNK), jnp.int32),
            pltpu.VMEM((_CPT, _CHUNK), jnp.int32),
            pltpu.VMEM((_CPT, _CHUNK), jnp.float32),
            pltpu.VMEM((_CPT, _CHUNK), jnp.float32),
            pltpu.VMEM((16,), jnp.float32),
            pltpu.VMEM((_NT, 16), jnp.float32),
            pltpu.SemaphoreType.DMA,
            pltpu.SemaphoreType.DMA,
        ],
    )
    _, res = f(outflat, indflat, maskflat, tgtflat)
    return (res[0, :_D] + res[1, :_D])


# speedup vs baseline: 1.2754x; 1.0559x over previous
import jax
import jax.numpy as jnp
from jax import lax
from jax.experimental import pallas as pl
from jax.experimental.pallas import tpu as pltpu
from jax.experimental.pallas import tpu_sc as plsc

_B, _D, _H, _W, _M = 8, 10, 188, 188, 500
_HW = _H * _W
_MP = 512
_NT = 16
_NC = 2
_NW = _NT * _NC
_CHUNK = 128
_NCHUNKS = _D * _B * (_MP // _CHUNK)
_CPT = _NCHUNKS // _NW
_NV = _CHUNK // 16


def _sc_loss_body(outflat, indflat, maskflat, tgtflat, part, out,
                  ind_v, mask_v, idxp_v, pred_v, tgt_v,
                  red_v, sum_v, psem, tsem):
    core = lax.axis_index("c")
    sub = lax.axis_index("s")
    wid = core * _NT + sub

    pltpu.sync_copy(indflat, ind_v)
    pltpu.sync_copy(maskflat, mask_v)

    lanes = lax.iota(jnp.int32, 16)

    handles = []
    for k in range(_CPT):
        c = wid * _CPT + k
        d = c // 32
        r = c % 32
        b = r // 4
        mc = r % 4
        ioff = b * _MP + mc * _CHUNK
        pbase = d * (_H * _B * _W)
        for j in range(_NV):
            iv = ind_v[pl.ds(ioff + j * 16, 16)]
            idxp_v[k, pl.ds(j * 16, 16)] = iv + pbase
        hp = pltpu.async_copy(outflat.at[idxp_v.at[k]], pred_v.at[k], psem)
        toff = (d * _B + b) * _MP + mc * _CHUNK
        ht = pltpu.async_copy(tgtflat.at[pl.ds(toff, _CHUNK)],
                              tgt_v.at[k], tsem)
        handles.append((hp, ht))

    for hp, ht in handles:
        hp.wait()
        ht.wait()

    acc = jnp.zeros((16,), jnp.float32)
    msum = jnp.float32(0.0)
    dfirst = core * (_D // _NC)
    for k in range(_CPT):
        c = wid * _CPT + k
        d = c // 32
        r = c % 32
        b = r // 4
        mc = r % 4
        ioff = b * _MP + mc * _CHUNK
        csum = jnp.zeros((16,), jnp.float32)
        mk = jnp.zeros((16,), jnp.float32)
        for j in range(_NV):
            p = pred_v[k, pl.ds(j * 16, 16)]
            t = tgt_v[k, pl.ds(j * 16, 16)]
            w = mask_v[pl.ds(ioff + j * 16, 16)]
            wm = jnp.where(t == t, w, jnp.float32(0.0))
            csum = csum + jnp.abs(p * wm - t * wm)
            mk = mk + w
        sval = jnp.sum(csum)
        msum = msum + jnp.where(d == dfirst, jnp.sum(mk), jnp.float32(0.0))
        acc = acc + jnp.where(lanes == d, sval, jnp.float32(0.0))

    acc = acc + jnp.where(lanes == _D, msum, jnp.float32(0.0))

    red_v[...] = acc
    pltpu.sync_copy(red_v, part.at[wid])
    plsc.subcore_barrier()

    @pl.when(sub == 0)
    def _final():
        pltpu.sync_copy(part.at[pl.ds(core * _NT, _NT)], sum_v)
        tot = jnp.zeros((16,), jnp.float32)
        for i in range(_NT):
            tot = tot + sum_v[i, :]
        num_v = jnp.full((16,), tot[_D], jnp.float32)
        denom = jnp.maximum(num_v, jnp.float32(1.0))
        red_v[...] = tot / denom
        pltpu.sync_copy(red_v, out.at[core])


def kernel(output, mask, ind, target):
    outflat = jnp.transpose(output, (1, 2, 0, 3)).reshape(-1)
    ind32 = ind.astype(jnp.int32)
    h = ind32 // _W
    w = ind32 - h * _W
    base = h * (_B * _W) + jnp.arange(_B, dtype=jnp.int32)[:, None] * _W + w
    indflat = jnp.pad(base, ((0, 0), (0, _MP - _M))).reshape(-1)
    maskflat = jnp.pad(mask.astype(jnp.float32),
                       ((0, 0), (0, _MP - _M))).reshape(-1)
    tgtflat = jnp.pad(jnp.transpose(target, (2, 0, 1)),
                      ((0, 0), (0, 0), (0, _MP - _M))).reshape(-1)

    mesh = plsc.VectorSubcoreMesh(core_axis_name="c", subcore_axis_name="s")
    f = pl.kernel(
        _sc_loss_body,
        out_type=(jax.ShapeDtypeStruct((_NW, 16), jnp.float32),
                  jax.ShapeDtypeStruct((_NC, 16), jnp.float32)),
        mesh=mesh,
        compiler_params=pltpu.CompilerParams(needs_layout_passes=False),
        scratch_types=[
            pltpu.VMEM((_B * _MP,), jnp.int32),
            pltpu.VMEM((_B * _MP,), jnp.float32),
            pltpu.VMEM((_CPT, _CHUNK), jnp.int32),
            pltpu.VMEM((_CPT, _CHUNK), jnp.float32),
            pltpu.VMEM((_CPT, _CHUNK), jnp.float32),
            pltpu.VMEM((16,), jnp.float32),
            pltpu.VMEM((_NT, 16), jnp.float32),
            pltpu.SemaphoreType.DMA,
            pltpu.SemaphoreType.DMA,
        ],
    )
    _, res = f(outflat, indflat, maskflat, tgtflat)
    return (res[0, :_D] + res[1, :_D])


# speedup vs baseline: 1.3370x; 1.0483x over previous
import jax
import jax.numpy as jnp
from jax import lax
from jax.experimental import pallas as pl
from jax.experimental.pallas import tpu as pltpu
from jax.experimental.pallas import tpu_sc as plsc

_B, _D, _H, _W, _M = 8, 10, 188, 188, 500
_HW = _H * _W
_MP = 512
_NT = 16
_NC = 2
_NW = _NT * _NC
_CHUNK = 128
_NCHUNKS = _D * _B * (_MP // _CHUNK)
_CPT = _NCHUNKS // _NW
_NV = _CHUNK // 16


def _sc_loss_body(outflat, encflat, tgtflat, part, out,
                  enc_v, idxp_v, pred_v, tgt_v,
                  red_v, sum_v, psem, tsem):
    core = lax.axis_index("c")
    sub = lax.axis_index("s")
    wid = core * _NT + sub

    pltpu.sync_copy(encflat, enc_v)

    lanes = lax.iota(jnp.int32, 16)

    handles = []
    for k in range(_CPT):
        c = wid * _CPT + k
        d = c // 32
        r = c % 32
        b = r // 4
        mc = r % 4
        ioff = b * _MP + mc * _CHUNK
        pbase = d * (_H * _B * _W)
        for j in range(_NV):
            iv = lax.shift_right_logical(enc_v[pl.ds(ioff + j * 16, 16)], 1)
            idxp_v[k, pl.ds(j * 16, 16)] = iv + pbase
        hp = pltpu.async_copy(outflat.at[idxp_v.at[k]], pred_v.at[k], psem)
        toff = (d * _B + b) * _MP + mc * _CHUNK
        ht = pltpu.async_copy(tgtflat.at[pl.ds(toff, _CHUNK)],
                              tgt_v.at[k], tsem)
        handles.append((hp, ht))

    for hp, ht in handles:
        hp.wait()
        ht.wait()

    acc = jnp.zeros((16,), jnp.float32)
    msum = jnp.float32(0.0)
    dfirst = core * (_D // _NC)
    for k in range(_CPT):
        c = wid * _CPT + k
        d = c // 32
        r = c % 32
        b = r // 4
        mc = r % 4
        ioff = b * _MP + mc * _CHUNK
        csum = jnp.zeros((16,), jnp.float32)
        mk = jnp.zeros((16,), jnp.float32)
        for j in range(_NV):
            p = pred_v[k, pl.ds(j * 16, 16)]
            t = tgt_v[k, pl.ds(j * 16, 16)]
            w = (enc_v[pl.ds(ioff + j * 16, 16)] & 1).astype(jnp.float32)
            wm = jnp.where(t == t, w, jnp.float32(0.0))
            csum = csum + jnp.abs(p * wm - t * wm)
            mk = mk + w
        sval = jnp.sum(csum)
        msum = msum + jnp.where(d == dfirst, jnp.sum(mk), jnp.float32(0.0))
        acc = acc + jnp.where(lanes == d, sval, jnp.float32(0.0))

    acc = acc + jnp.where(lanes == _D, msum, jnp.float32(0.0))

    red_v[...] = acc
    pltpu.sync_copy(red_v, part.at[wid])
    plsc.subcore_barrier()

    @pl.when(sub == 0)
    def _final():
        pltpu.sync_copy(part.at[pl.ds(core * _NT, _NT)], sum_v)
        tot = jnp.zeros((16,), jnp.float32)
        for i in range(_NT):
            tot = tot + sum_v[i, :]
        num_v = jnp.full((16,), tot[_D], jnp.float32)
        denom = jnp.maximum(num_v, jnp.float32(1.0))
        red_v[...] = tot / denom
        pltpu.sync_copy(red_v, out.at[core])


def kernel(output, mask, ind, target):
    outflat = jnp.transpose(output, (1, 2, 0, 3)).reshape(-1)
    ind32 = ind.astype(jnp.int32)
    h = ind32 // _W
    w = ind32 - h * _W
    base = h * (_B * _W) + jnp.arange(_B, dtype=jnp.int32)[:, None] * _W + w
    enc = (base << 1) | (mask.astype(jnp.int32) & 1)
    encflat = jnp.pad(enc, ((0, 0), (0, _MP - _M))).reshape(-1)
    tgtflat = jnp.pad(jnp.transpose(target, (2, 0, 1)),
                      ((0, 0), (0, 0), (0, _MP - _M))).reshape(-1)

    mesh = plsc.VectorSubcoreMesh(core_axis_name="c", subcore_axis_name="s")
    f = pl.kernel(
        _sc_loss_body,
        out_type=(jax.ShapeDtypeStruct((_NW, 16), jnp.float32),
                  jax.ShapeDtypeStruct((_NC, 16), jnp.float32)),
        mesh=mesh,
        compiler_params=pltpu.CompilerParams(needs_layout_passes=False),
        scratch_types=[
            pltpu.VMEM((_B * _MP,), jnp.int32),
            pltpu.VMEM((_CPT, _CHUNK), jnp.int32),
            pltpu.VMEM((_CPT, _CHUNK), jnp.float32),
            pltpu.VMEM((_CPT, _CHUNK), jnp.float32),
            pltpu.VMEM((16,), jnp.float32),
            pltpu.VMEM((_NT, 16), jnp.float32),
            pltpu.SemaphoreType.DMA,
            pltpu.SemaphoreType.DMA,
        ],
    )
    _, res = f(outflat, encflat, tgtflat)
    return (res[0, :_D] + res[1, :_D])
